# baseline (speedup 1.0000x reference)
import functools

import jax
import jax.numpy as jnp
from jax import lax
from jax.experimental import pallas as pl
from jax.experimental.pallas import tpu as pltpu
from jax.experimental.pallas import tpu_sc as plsc

D = 768
H = 2048
E = 8
S = 2048
K = 2
A = K * S
BLK = 128
CAP = A + E * BLK
NBLK = CAP // BLK
NC = 2
NS = 16
NW = NC * NS
RPW = CAP // NW
TPW = S // NW
GCH = 32


def _route_kernel(x_ref, wg_ref, bg_ref, dest_ref, va_ref, bexp_ref):
    xb = x_ref[...]
    logits = jnp.dot(xb, wg_ref[...], preferred_element_type=jnp.float32)
    logits = logits + bg_ref[...]
    m = jnp.max(logits, axis=-1, keepdims=True)
    ex = jnp.exp(logits - m)
    rw = ex / jnp.sum(ex, axis=-1, keepdims=True)
    ii = lax.broadcasted_iota(jnp.int32, (S, E), 1)
    m1 = jnp.max(rw, axis=-1, keepdims=True)
    e1 = jnp.min(jnp.where(rw == m1, ii, E), axis=-1, keepdims=True)
    rw2 = jnp.where(ii == e1, -1.0, rw)
    m2 = jnp.max(rw2, axis=-1, keepdims=True)
    e2 = jnp.min(jnp.where(rw2 == m2, ii, E), axis=-1, keepdims=True)

    ea = jnp.concatenate([e1, e2], axis=0)
    va = jnp.concatenate([m1, m2], axis=0)
    va_ref[...] = va

    iiE = lax.broadcasted_iota(jnp.int32, (A, E), 1)
    oh = (ea == iiE).astype(jnp.float32)
    cum = oh
    s = 1
    while s < A:
        cum = cum + jnp.concatenate(
            [jnp.zeros((s, E), jnp.float32), cum[:-s, :]], axis=0)
        s *= 2
    rank = jnp.sum(oh * cum, axis=-1, keepdims=True) - 1.0
    count = cum[A - 1:A, :]
    co = jnp.floor((count + (BLK - 1)) * (1.0 / BLK)) * BLK
    off = jnp.concatenate([jnp.zeros((1, 1), jnp.float32), co[:, :-1]],
                          axis=1)
    s = 1
    while s < E:
        off = off + jnp.concatenate(
            [jnp.zeros((1, s), jnp.float32), off[:, :-s]], axis=1)
        s *= 2
    offsel = jnp.sum(oh * off, axis=-1, keepdims=True)
    dest_ref[...] = (offsel + rank).astype(jnp.int32)

    bb = lax.broadcasted_iota(jnp.int32, (1, NBLK), 1).astype(jnp.float32)
    bb = bb * BLK
    bexpf = jnp.zeros((1, NBLK), jnp.float32)
    for e in range(E):
        bexpf = bexpf + (bb >= off[:, e:e + 1]).astype(jnp.float32)
    bexp_ref[...] = (bexpf - 1.0).astype(jnp.int32)


@jax.jit
def _route(x2, Wg, bg):
    return pl.pallas_call(
        _route_kernel,
        in_specs=[
            pl.BlockSpec((S, D), lambda: (0, 0)),
            pl.BlockSpec((D, E), lambda: (0, 0)),
            pl.BlockSpec((1, E), lambda: (0, 0)),
        ],
        out_specs=[
            pl.BlockSpec((A, 1), lambda: (0, 0)),
            pl.BlockSpec((A, 1), lambda: (0, 0)),
            pl.BlockSpec((1, NBLK), lambda: (0, 0)),
        ],
        out_shape=[
            jax.ShapeDtypeStruct((A, 1), jnp.int32),
            jax.ShapeDtypeStruct((A, 1), jnp.float32),
            jax.ShapeDtypeStruct((1, NBLK), jnp.int32),
        ],
    )(x2, Wg, bg)


def _sc_dispatch_body(dest_hbm, va_hbm, x_hbm, sx_hbm, sw_hbm,
                      dest_v, va_v, stok_v, sw_v, idx_v, rows_v, sem):
    cid = lax.axis_index("c")
    sid = lax.axis_index("s")
    wid = sid * NC + cid

    pltpu.sync_copy(dest_hbm, dest_v)
    pltpu.sync_copy(va_hbm, va_v)

    zero_i = jnp.zeros((16,), jnp.int32)
    zero_f = jnp.zeros((16,), jnp.float32)

    def zbody(i, c):
        stok_v[pl.ds(i * 16, 16)] = zero_i
        sw_v[pl.ds(i * 16, 16)] = zero_f
        return c

    lax.fori_loop(0, CAP // 16, zbody, 0)

    lane = lax.broadcasted_iota(jnp.int32, (16,), 0)

    def sbody(c, carry):
        base = c * 16
        av = dest_v[pl.ds(base, 16)]
        tok = jnp.bitwise_and(base + lane, S - 1)
        vv = va_v[pl.ds(base, 16)]
        plsc.store_scatter(stok_v, [av], tok)
        plsc.store_scatter(sw_v, [av], vv)
        return carry

    lax.fori_loop(0, A // 16, sbody, 0)

    @pl.when(wid == 0)
    def _():
        pltpu.sync_copy(sw_v, sw_hbm)

    for ch in range(RPW // GCH):
        rbase = wid * RPW + ch * GCH
        idx_v[pl.ds(0, 16)] = stok_v[pl.ds(rbase, 16)]
        idx_v[pl.ds(16, 16)] = stok_v[pl.ds(rbase + 16, 16)]
        pltpu.async_copy(x_hbm.at[idx_v], rows_v, sem).wait()
        pltpu.sync_copy(rows_v, sx_hbm.at[pl.ds(rbase, GCH)])


@functools.cache
def _get_sc_dispatch():
    return pl.kernel(
        _sc_dispatch_body,
        out_type=[
            jax.ShapeDtypeStruct((CAP, D), jnp.float32),
            jax.ShapeDtypeStruct((CAP,), jnp.float32),
        ],
        mesh=plsc.VectorSubcoreMesh(core_axis_name="c",
                                    subcore_axis_name="s"),
        compiler_params=pltpu.CompilerParams(needs_layout_passes=False),
        scratch_types=[
            pltpu.VMEM((A,), jnp.int32),
            pltpu.VMEM((A,), jnp.float32),
            pltpu.VMEM((CAP,), jnp.int32),
            pltpu.VMEM((CAP,), jnp.float32),
            pltpu.VMEM((GCH,), jnp.int32),
            pltpu.VMEM((GCH, D), jnp.float32),
            pltpu.SemaphoreType.DMA,
        ],
    )


def _ffn_kernel(bexp_ref, sx_ref, sw_ref, w1_ref, b1_ref, w2_ref, b2_ref,
                out_ref):
    xb = sx_ref[...]
    h = jnp.dot(xb, w1_ref[0], preferred_element_type=jnp.float32)
    h = jnp.maximum(h + b1_ref[0], 0.0)
    o = jnp.dot(h, w2_ref[0], preferred_element_type=jnp.float32)
    o = o + b2_ref[0]
    out_ref[...] = o * sw_ref[...]


@jax.jit
def _ffn(bexp, sx, sw2, W1, b1, W2, b2):
    grid_spec = pltpu.PrefetchScalarGridSpec(
        num_scalar_prefetch=1,
        grid=(NBLK,),
        in_specs=[
            pl.BlockSpec((BLK, D), lambda i, be: (i, 0)),
            pl.BlockSpec((BLK, 1), lambda i, be: (i, 0)),
            pl.BlockSpec((1, D, H), lambda i, be: (be[i], 0, 0)),
            pl.BlockSpec((1, 1, H), lambda i, be: (be[i], 0, 0)),
            pl.BlockSpec((1, H, D), lambda i, be: (be[i], 0, 0)),
            pl.BlockSpec((1, 1, D), lambda i, be: (be[i], 0, 0)),
        ],
        out_specs=pl.BlockSpec((BLK, D), lambda i, be: (i, 0)),
    )
    return pl.pallas_call(
        _ffn_kernel,
        grid_spec=grid_spec,
        out_shape=jax.ShapeDtypeStruct((CAP, D), jnp.float32),
        compiler_params=pltpu.CompilerParams(
            dimension_semantics=("arbitrary",)),
    )(bexp, sx, sw2, W1, b1, W2, b2)


def _sc_combine_body(dest_hbm, ffn_hbm, x_hbm, y_hbm,
                     idx1_v, idx2_v, g1_v, g2_v, xb_v, sem):
    cid = lax.axis_index("c")
    sid = lax.axis_index("s")
    wid = sid * NC + cid

    for hf in range(2):
        tb = wid * TPW + hf * 32
        pltpu.sync_copy(dest_hbm.at[pl.ds(tb, 32)], idx1_v)
        pltpu.sync_copy(dest_hbm.at[pl.ds(S + tb, 32)], idx2_v)
        pltpu.async_copy(ffn_hbm.at[idx1_v], g1_v, sem).wait()
        pltpu.async_copy(ffn_hbm.at[idx2_v], g2_v, sem).wait()
        pltpu.sync_copy(x_hbm.at[pl.ds(tb, 32)], xb_v)

        def cbody(r, carry):
            for cc in range(D // 16):
                sl = pl.ds(cc * 16, 16)
                g1_v[r, sl] = g1_v[r, sl] + g2_v[r, sl] + xb_v[r, sl]
            return carry

        lax.fori_loop(0, 32, cbody, 0)
        pltpu.sync_copy(g1_v, y_hbm.at[pl.ds(tb, 32)])


@functools.cache
def _get_sc_combine():
    return pl.kernel(
        _sc_combine_body,
        out_type=jax.ShapeDtypeStruct((S, D), jnp.float32),
        mesh=plsc.VectorSubcoreMesh(core_axis_name="c",
                                    subcore_axis_name="s"),
        compiler_params=pltpu.CompilerParams(needs_layout_passes=False),
        scratch_types=[
            pltpu.VMEM((32,), jnp.int32),
            pltpu.VMEM((32,), jnp.int32),
            pltpu.VMEM((32, D), jnp.float32),
            pltpu.VMEM((32, D), jnp.float32),
            pltpu.VMEM((32, D), jnp.float32),
            pltpu.SemaphoreType.DMA,
        ],
    )


LBLK = 512


def _ln_kernel(y_ref, gamma_ref, beta_ref, out_ref):
    y = y_ref[...]
    mean = jnp.mean(y, axis=-1, keepdims=True)
    c = y - mean
    var = jnp.mean(c * c, axis=-1, keepdims=True)
    out_ref[...] = (c * lax.rsqrt(var + 1e-5) * gamma_ref[...]
                    + beta_ref[...])


@jax.jit
def _ln(y, gamma, beta):
    return pl.pallas_call(
        _ln_kernel,
        grid=(S // LBLK,),
        in_specs=[
            pl.BlockSpec((LBLK, D), lambda i: (i, 0)),
            pl.BlockSpec((1, D), lambda i: (0, 0)),
            pl.BlockSpec((1, D), lambda i: (0, 0)),
        ],
        out_specs=pl.BlockSpec((LBLK, D), lambda i: (i, 0)),
        out_shape=jax.ShapeDtypeStruct((S, D), jnp.float32),
    )(y, gamma, beta)


def kernel(x, Wg, bg, W1, b1, W2, b2, gamma, beta):
    x2 = x.reshape(S, D)
    dest2, va2, bexp2 = _route(x2, Wg, bg.reshape(1, E))
    dest = dest2.reshape(A)
    va = va2.reshape(A)
    bexp = bexp2.reshape(NBLK)
    sx, sw = _get_sc_dispatch()(dest, va, x2)
    ffn = _ffn(bexp, sx, sw.reshape(CAP, 1), W1, b1.reshape(E, 1, H),
               W2, b2.reshape(E, 1, D))
    y = _get_sc_combine()(dest, ffn, x2)
    out = _ln(y, gamma.reshape(1, D), beta.reshape(1, D))
    return out.reshape(x.shape)


# speedup vs baseline: 1.0486x; 1.0486x over previous
import functools

import jax
import jax.numpy as jnp
from jax import lax
from jax.experimental import pallas as pl
from jax.experimental.pallas import tpu as pltpu
from jax.experimental.pallas import tpu_sc as plsc

D = 768
H = 2048
E = 8
S = 2048
K = 2
A = K * S
BLK = 128
CAP = A + E * BLK
NBLK = CAP // BLK
NC = 2
NS = 16
NW = NC * NS
RPW = CAP // NW
TPW = S // NW
GCH = 32


def _route_kernel(x_ref, wg_ref, bg_ref, dest_ref, va_ref, bexp_ref):
    xb = x_ref[...]
    logits = jnp.dot(xb, wg_ref[...], preferred_element_type=jnp.float32)
    logits = logits + bg_ref[...]
    m = jnp.max(logits, axis=-1, keepdims=True)
    ex = jnp.exp(logits - m)
    rw = ex / jnp.sum(ex, axis=-1, keepdims=True)
    ii = lax.broadcasted_iota(jnp.int32, (S, E), 1)
    m1 = jnp.max(rw, axis=-1, keepdims=True)
    e1 = jnp.min(jnp.where(rw == m1, ii, E), axis=-1, keepdims=True)
    rw2 = jnp.where(ii == e1, -1.0, rw)
    m2 = jnp.max(rw2, axis=-1, keepdims=True)
    e2 = jnp.min(jnp.where(rw2 == m2, ii, E), axis=-1, keepdims=True)

    ea = jnp.concatenate([e1, e2], axis=0)
    va = jnp.concatenate([m1, m2], axis=0)
    va_ref[...] = va

    iiE = lax.broadcasted_iota(jnp.int32, (A, E), 1)
    oh = (ea == iiE).astype(jnp.float32)
    cum = oh
    s = 1
    while s < A:
        cum = cum + jnp.concatenate(
            [jnp.zeros((s, E), jnp.float32), cum[:-s, :]], axis=0)
        s *= 2
    rank = jnp.sum(oh * cum, axis=-1, keepdims=True) - 1.0
    count = cum[A - 1:A, :]
    co = jnp.floor((count + (BLK - 1)) * (1.0 / BLK)) * BLK
    off = jnp.concatenate([jnp.zeros((1, 1), jnp.float32), co[:, :-1]],
                          axis=1)
    s = 1
    while s < E:
        off = off + jnp.concatenate(
            [jnp.zeros((1, s), jnp.float32), off[:, :-s]], axis=1)
        s *= 2
    offsel = jnp.sum(oh * off, axis=-1, keepdims=True)
    dest_ref[...] = (offsel + rank).astype(jnp.int32)

    bb = lax.broadcasted_iota(jnp.int32, (1, NBLK), 1).astype(jnp.float32)
    bb = bb * BLK
    bexpf = jnp.zeros((1, NBLK), jnp.float32)
    for e in range(E):
        bexpf = bexpf + (bb >= off[:, e:e + 1]).astype(jnp.float32)
    bexp_ref[...] = (bexpf - 1.0).astype(jnp.int32)


@jax.jit
def _route(x2, Wg, bg):
    return pl.pallas_call(
        _route_kernel,
        in_specs=[
            pl.BlockSpec((S, D), lambda: (0, 0)),
            pl.BlockSpec((D, E), lambda: (0, 0)),
            pl.BlockSpec((1, E), lambda: (0, 0)),
        ],
        out_specs=[
            pl.BlockSpec((A, 1), lambda: (0, 0)),
            pl.BlockSpec((A, 1), lambda: (0, 0)),
            pl.BlockSpec((1, NBLK), lambda: (0, 0)),
        ],
        out_shape=[
            jax.ShapeDtypeStruct((A, 1), jnp.int32),
            jax.ShapeDtypeStruct((A, 1), jnp.float32),
            jax.ShapeDtypeStruct((1, NBLK), jnp.int32),
        ],
    )(x2, Wg, bg)


def _sc_dispatch_body(dest_hbm, va_hbm, x_hbm, sx_hbm, sw_hbm,
                      dest_v, va_v, stok_v, sw_v,
                      idx0_v, idx1_v, rows0_v, rows1_v,
                      isem, g0, g1, w0, w1):
    cid = lax.axis_index("c")
    sid = lax.axis_index("s")
    wid = sid * NC + cid

    cin0 = pltpu.async_copy(dest_hbm, dest_v, isem)
    cin1 = pltpu.async_copy(va_hbm, va_v, isem)

    zero_i = jnp.zeros((16,), jnp.int32)
    zero_f = jnp.zeros((16,), jnp.float32)

    def zbody(i, c):
        stok_v[pl.ds(i * 16, 16)] = zero_i
        sw_v[pl.ds(i * 16, 16)] = zero_f
        return c

    lax.fori_loop(0, CAP // 16, zbody, 0)
    cin0.wait()
    cin1.wait()

    lane = lax.broadcasted_iota(jnp.int32, (16,), 0)

    def sbody(c, carry):
        base = c * 16
        av = dest_v[pl.ds(base, 16)]
        tok = jnp.bitwise_and(base + lane, S - 1)
        vv = va_v[pl.ds(base, 16)]
        plsc.store_scatter(stok_v, [av], tok)
        plsc.store_scatter(sw_v, [av], vv)
        return carry

    lax.fori_loop(0, A // 16, sbody, 0)

    @pl.when(wid == 0)
    def _():
        pltpu.sync_copy(sw_v, sw_hbm)

    idx = (idx0_v, idx1_v)
    rows = (rows0_v, rows1_v)
    gsem = (g0, g1)
    wsem = (w0, w1)
    nch = RPW // GCH
    gh = [None] * nch
    wh = [None] * nch
    for ch in range(nch):
        b = ch % 2
        rbase = wid * RPW + ch * GCH
        idx[b][pl.ds(0, 16)] = stok_v[pl.ds(rbase, 16)]
        idx[b][pl.ds(16, 16)] = stok_v[pl.ds(rbase + 16, 16)]
        if ch >= 2:
            wh[ch - 2].wait()
        gh[ch] = pltpu.async_copy(x_hbm.at[idx[b]], rows[b], gsem[b])
        if ch >= 1:
            gh[ch - 1].wait()
            pb = (ch - 1) % 2
            pbase = wid * RPW + (ch - 1) * GCH
            wh[ch - 1] = pltpu.async_copy(
                rows[pb], sx_hbm.at[pl.ds(pbase, GCH)], wsem[pb])
    gh[nch - 1].wait()
    lb = (nch - 1) % 2
    lbase = wid * RPW + (nch - 1) * GCH
    wh[nch - 1] = pltpu.async_copy(
        rows[lb], sx_hbm.at[pl.ds(lbase, GCH)], wsem[lb])
    wh[nch - 2].wait()
    wh[nch - 1].wait()


@functools.cache
def _get_sc_dispatch():
    return pl.kernel(
        _sc_dispatch_body,
        out_type=[
            jax.ShapeDtypeStruct((CAP, D), jnp.float32),
            jax.ShapeDtypeStruct((CAP,), jnp.float32),
        ],
        mesh=plsc.VectorSubcoreMesh(core_axis_name="c",
                                    subcore_axis_name="s"),
        compiler_params=pltpu.CompilerParams(needs_layout_passes=False),
        scratch_types=[
            pltpu.VMEM((A,), jnp.int32),
            pltpu.VMEM((A,), jnp.float32),
            pltpu.VMEM((CAP,), jnp.int32),
            pltpu.VMEM((CAP,), jnp.float32),
            pltpu.VMEM((GCH,), jnp.int32),
            pltpu.VMEM((GCH,), jnp.int32),
            pltpu.VMEM((GCH, D), jnp.float32),
            pltpu.VMEM((GCH, D), jnp.float32),
            pltpu.SemaphoreType.DMA,
            pltpu.SemaphoreType.DMA,
            pltpu.SemaphoreType.DMA,
            pltpu.SemaphoreType.DMA,
            pltpu.SemaphoreType.DMA,
        ],
    )


def _ffn_kernel(bexp_ref, sx_ref, sw_ref, w1_ref, b1_ref, w2_ref, b2_ref,
                out_ref):
    xb = sx_ref[...]
    h = jnp.dot(xb, w1_ref[0], preferred_element_type=jnp.float32)
    h = jnp.maximum(h + b1_ref[0], 0.0)
    o = jnp.dot(h, w2_ref[0], preferred_element_type=jnp.float32)
    o = o + b2_ref[0]
    out_ref[...] = o * sw_ref[...]


@jax.jit
def _ffn(bexp, sx, sw2, W1, b1, W2, b2):
    grid_spec = pltpu.PrefetchScalarGridSpec(
        num_scalar_prefetch=1,
        grid=(NBLK,),
        in_specs=[
            pl.BlockSpec((BLK, D), lambda i, be: (i, 0)),
            pl.BlockSpec((BLK, 1), lambda i, be: (i, 0)),
            pl.BlockSpec((1, D, H), lambda i, be: (be[i], 0, 0)),
            pl.BlockSpec((1, 1, H), lambda i, be: (be[i], 0, 0)),
            pl.BlockSpec((1, H, D), lambda i, be: (be[i], 0, 0)),
            pl.BlockSpec((1, 1, D), lambda i, be: (be[i], 0, 0)),
        ],
        out_specs=pl.BlockSpec((BLK, D), lambda i, be: (i, 0)),
    )
    return pl.pallas_call(
        _ffn_kernel,
        grid_spec=grid_spec,
        out_shape=jax.ShapeDtypeStruct((CAP, D), jnp.float32),
        compiler_params=pltpu.CompilerParams(
            dimension_semantics=("arbitrary",)),
    )(bexp, sx, sw2, W1, b1, W2, b2)


def _sc_combine_body(dest_hbm, ffn_hbm, y1_hbm, y2_hbm,
                     idx1_v, idx2_v, g1a_v, g1b_v, g2a_v, g2b_v,
                     s0, s1, s2, s3):
    cid = lax.axis_index("c")
    sid = lax.axis_index("s")
    wid = sid * NC + cid
    tb = wid * TPW

    ci1 = pltpu.async_copy(dest_hbm.at[pl.ds(tb, TPW)], idx1_v, s0)
    ci2 = pltpu.async_copy(dest_hbm.at[pl.ds(S + tb, TPW)], idx2_v, s1)
    ci1.wait()
    ci2.wait()

    gs = []
    bufs = ((g1a_v, idx1_v, 0, y1_hbm, s0), (g2a_v, idx2_v, 0, y2_hbm, s1),
            (g1b_v, idx1_v, 32, y1_hbm, s2), (g2b_v, idx2_v, 32, y2_hbm, s3))
    for buf, idxv, hoff, _, sem in bufs:
        gs.append(pltpu.async_copy(ffn_hbm.at[idxv.at[pl.ds(hoff, 32)]],
                                   buf, sem))
    ws = []
    for i, (buf, _, hoff, yhbm, sem) in enumerate(bufs):
        gs[i].wait()
        ws.append(pltpu.async_copy(buf, yhbm.at[pl.ds(tb + hoff, 32)], sem))
    for w in ws:
        w.wait()


@functools.cache
def _get_sc_combine():
    return pl.kernel(
        _sc_combine_body,
        out_type=[
            jax.ShapeDtypeStruct((S, D), jnp.float32),
            jax.ShapeDtypeStruct((S, D), jnp.float32),
        ],
        mesh=plsc.VectorSubcoreMesh(core_axis_name="c",
                                    subcore_axis_name="s"),
        compiler_params=pltpu.CompilerParams(needs_layout_passes=False),
        scratch_types=[
            pltpu.VMEM((TPW,), jnp.int32),
            pltpu.VMEM((TPW,), jnp.int32),
            pltpu.VMEM((32, D), jnp.float32),
            pltpu.VMEM((32, D), jnp.float32),
            pltpu.VMEM((32, D), jnp.float32),
            pltpu.VMEM((32, D), jnp.float32),
            pltpu.SemaphoreType.DMA,
            pltpu.SemaphoreType.DMA,
            pltpu.SemaphoreType.DMA,
            pltpu.SemaphoreType.DMA,
        ],
    )


LBLK = 512


def _ln_kernel(y1_ref, y2_ref, x_ref, gamma_ref, beta_ref, out_ref):
    y = y1_ref[...] + y2_ref[...] + x_ref[...]
    mean = jnp.mean(y, axis=-1, keepdims=True)
    c = y - mean
    var = jnp.mean(c * c, axis=-1, keepdims=True)
    out_ref[...] = (c * lax.rsqrt(var + 1e-5) * gamma_ref[...]
                    + beta_ref[...])


@jax.jit
def _ln(y1, y2, x2, gamma, beta):
    return pl.pallas_call(
        _ln_kernel,
        grid=(S // LBLK,),
        in_specs=[
            pl.BlockSpec((LBLK, D), lambda i: (i, 0)),
            pl.BlockSpec((LBLK, D), lambda i: (i, 0)),
            pl.BlockSpec((LBLK, D), lambda i: (i, 0)),
            pl.BlockSpec((1, D), lambda i: (0, 0)),
            pl.BlockSpec((1, D), lambda i: (0, 0)),
        ],
        out_specs=pl.BlockSpec((LBLK, D), lambda i: (i, 0)),
        out_shape=jax.ShapeDtypeStruct((S, D), jnp.float32),
    )(y1, y2, x2, gamma, beta)


def kernel(x, Wg, bg, W1, b1, W2, b2, gamma, beta):
    x2 = x.reshape(S, D)
    dest2, va2, bexp2 = _route(x2, Wg, bg.reshape(1, E))
    dest = dest2.reshape(A)
    va = va2.reshape(A)
    bexp = bexp2.reshape(NBLK)
    sx, sw = _get_sc_dispatch()(dest, va, x2)
    ffn = _ffn(bexp, sx, sw.reshape(CAP, 1), W1, b1.reshape(E, 1, H),
               W2, b2.reshape(E, 1, D))
    y1, y2 = _get_sc_combine()(dest, ffn)
    out = _ln(y1, y2, x2, gamma.reshape(1, D), beta.reshape(1, D))
    return out.reshape(x.shape)


# speedup vs baseline: 1.0800x; 1.0300x over previous
import functools

import jax
import jax.numpy as jnp
from jax import lax
from jax.experimental import pallas as pl
from jax.experimental.pallas import tpu as pltpu
from jax.experimental.pallas import tpu_sc as plsc

D = 768
H = 2048
E = 8
S = 2048
K = 2
A = K * S
BLK = 128
CAP = A + E * BLK
NBLK = CAP // BLK
NC = 2
NS = 16
NW = NC * NS
RPW = CAP // NW
TPW = S // NW
GCH = 32


def _route_kernel(x_ref, wg_ref, bg_ref, dest_ref, va_ref, bexp_ref):
    xb = x_ref[...]
    logits = jnp.dot(xb, wg_ref[...], preferred_element_type=jnp.float32)
    logits = logits + bg_ref[...]
    m = jnp.max(logits, axis=-1, keepdims=True)
    ex = jnp.exp(logits - m)
    rw = ex / jnp.sum(ex, axis=-1, keepdims=True)
    ii = lax.broadcasted_iota(jnp.int32, (S, E), 1)
    m1 = jnp.max(rw, axis=-1, keepdims=True)
    e1 = jnp.min(jnp.where(rw == m1, ii, E), axis=-1, keepdims=True)
    rw2 = jnp.where(ii == e1, -1.0, rw)
    m2 = jnp.max(rw2, axis=-1, keepdims=True)
    e2 = jnp.min(jnp.where(rw2 == m2, ii, E), axis=-1, keepdims=True)

    ea = jnp.concatenate([e1, e2], axis=0)
    va = jnp.concatenate([m1, m2], axis=0)
    va_ref[...] = va

    iiE = lax.broadcasted_iota(jnp.int32, (A, E), 1)
    oh = (ea == iiE).astype(jnp.float32)
    cum = oh
    s = 1
    while s < A:
        cum = cum + jnp.concatenate(
            [jnp.zeros((s, E), jnp.float32), cum[:-s, :]], axis=0)
        s *= 2
    rank = jnp.sum(oh * cum, axis=-1, keepdims=True) - 1.0
    count = cum[A - 1:A, :]
    co = jnp.floor((count + (BLK - 1)) * (1.0 / BLK)) * BLK
    off = jnp.concatenate([jnp.zeros((1, 1), jnp.float32), co[:, :-1]],
                          axis=1)
    s = 1
    while s < E:
        off = off + jnp.concatenate(
            [jnp.zeros((1, s), jnp.float32), off[:, :-s]], axis=1)
        s *= 2
    offsel = jnp.sum(oh * off, axis=-1, keepdims=True)
    dest_ref[...] = (offsel + rank).astype(jnp.int32)

    bb = lax.broadcasted_iota(jnp.int32, (1, NBLK), 1).astype(jnp.float32)
    bb = bb * BLK
    bexpf = jnp.zeros((1, NBLK), jnp.float32)
    for e in range(E):
        bexpf = bexpf + (bb >= off[:, e:e + 1]).astype(jnp.float32)
    bexp_ref[...] = (bexpf - 1.0).astype(jnp.int32)


@jax.jit
def _route(x2, Wg, bg):
    return pl.pallas_call(
        _route_kernel,
        in_specs=[
            pl.BlockSpec((S, D), lambda: (0, 0)),
            pl.BlockSpec((D, E), lambda: (0, 0)),
            pl.BlockSpec((1, E), lambda: (0, 0)),
        ],
        out_specs=[
            pl.BlockSpec((A, 1), lambda: (0, 0)),
            pl.BlockSpec((A, 1), lambda: (0, 0)),
            pl.BlockSpec((1, NBLK), lambda: (0, 0)),
        ],
        out_shape=[
            jax.ShapeDtypeStruct((A, 1), jnp.int32),
            jax.ShapeDtypeStruct((A, 1), jnp.float32),
            jax.ShapeDtypeStruct((1, NBLK), jnp.int32),
        ],
    )(x2, Wg, bg)


def _sc_dispatch_body(dest_hbm, x_hbm, sx_hbm,
                      dest_v, stok_v,
                      idx0_v, idx1_v, rows0_v, rows1_v,
                      isem, g0, g1, w0, w1):
    cid = lax.axis_index("c")
    sid = lax.axis_index("s")
    wid = sid * NC + cid

    cin0 = pltpu.async_copy(dest_hbm, dest_v, isem)

    zero_i = jnp.zeros((16,), jnp.int32)
    for i in range(RPW // 16):
        stok_v[pl.ds(wid * RPW + i * 16, 16)] = zero_i
    cin0.wait()

    lane = lax.broadcasted_iota(jnp.int32, (16,), 0)

    @plsc.parallel_loop(0, A // 16, unroll=8)
    def _(c):
        base = c * 16
        av = dest_v[pl.ds(base, 16)]
        tok = jnp.bitwise_and(base + lane, S - 1)
        plsc.store_scatter(stok_v, [av], tok)

    idx = (idx0_v, idx1_v)
    rows = (rows0_v, rows1_v)
    gsem = (g0, g1)
    wsem = (w0, w1)
    nch = RPW // GCH
    gh = [None] * nch
    wh = [None] * nch
    for ch in range(nch):
        b = ch % 2
        rbase = wid * RPW + ch * GCH
        idx[b][pl.ds(0, 16)] = stok_v[pl.ds(rbase, 16)]
        idx[b][pl.ds(16, 16)] = stok_v[pl.ds(rbase + 16, 16)]
        if ch >= 2:
            wh[ch - 2].wait()
        gh[ch] = pltpu.async_copy(x_hbm.at[idx[b]], rows[b], gsem[b])
        if ch >= 1:
            gh[ch - 1].wait()
            pb = (ch - 1) % 2
            pbase = wid * RPW + (ch - 1) * GCH
            wh[ch - 1] = pltpu.async_copy(
                rows[pb], sx_hbm.at[pl.ds(pbase, GCH)], wsem[pb])
    gh[nch - 1].wait()
    lb = (nch - 1) % 2
    lbase = wid * RPW + (nch - 1) * GCH
    wh[nch - 1] = pltpu.async_copy(
        rows[lb], sx_hbm.at[pl.ds(lbase, GCH)], wsem[lb])
    wh[nch - 2].wait()
    wh[nch - 1].wait()


@functools.cache
def _get_sc_dispatch():
    return pl.kernel(
        _sc_dispatch_body,
        out_type=jax.ShapeDtypeStruct((CAP, D), jnp.float32),
        mesh=plsc.VectorSubcoreMesh(core_axis_name="c",
                                    subcore_axis_name="s"),
        compiler_params=pltpu.CompilerParams(needs_layout_passes=False),
        scratch_types=[
            pltpu.VMEM((A,), jnp.int32),
            pltpu.VMEM((CAP,), jnp.int32),
            pltpu.VMEM((GCH,), jnp.int32),
            pltpu.VMEM((GCH,), jnp.int32),
            pltpu.VMEM((GCH, D), jnp.float32),
            pltpu.VMEM((GCH, D), jnp.float32),
            pltpu.SemaphoreType.DMA,
            pltpu.SemaphoreType.DMA,
            pltpu.SemaphoreType.DMA,
            pltpu.SemaphoreType.DMA,
            pltpu.SemaphoreType.DMA,
        ],
    )


def _ffn_kernel(bexp_ref, sx_ref, w1_ref, b1_ref, w2_ref, b2_ref,
                out_ref):
    xb = sx_ref[...]
    h = jnp.dot(xb, w1_ref[0], preferred_element_type=jnp.float32)
    h = jnp.maximum(h + b1_ref[0], 0.0)
    o = jnp.dot(h, w2_ref[0], preferred_element_type=jnp.float32)
    out_ref[...] = o + b2_ref[0]


@jax.jit
def _ffn(bexp, sx, W1, b1, W2, b2):
    grid_spec = pltpu.PrefetchScalarGridSpec(
        num_scalar_prefetch=1,
        grid=(NBLK,),
        in_specs=[
            pl.BlockSpec((BLK, D), lambda i, be: (i, 0)),
            pl.BlockSpec((1, D, H), lambda i, be: (be[i], 0, 0)),
            pl.BlockSpec((1, 1, H), lambda i, be: (be[i], 0, 0)),
            pl.BlockSpec((1, H, D), lambda i, be: (be[i], 0, 0)),
            pl.BlockSpec((1, 1, D), lambda i, be: (be[i], 0, 0)),
        ],
        out_specs=pl.BlockSpec((BLK, D), lambda i, be: (i, 0)),
    )
    return pl.pallas_call(
        _ffn_kernel,
        grid_spec=grid_spec,
        out_shape=jax.ShapeDtypeStruct((CAP, D), jnp.float32),
        compiler_params=pltpu.CompilerParams(
            dimension_semantics=("arbitrary",)),
    )(bexp, sx, W1, b1, W2, b2)


def _sc_combine_body(dest_hbm, ffn_hbm, y1_hbm, y2_hbm,
                     idx1_v, idx2_v, g1a_v, g1b_v, g2a_v, g2b_v,
                     s0, s1, s2, s3):
    cid = lax.axis_index("c")
    sid = lax.axis_index("s")
    wid = sid * NC + cid
    tb = wid * TPW

    ci1 = pltpu.async_copy(dest_hbm.at[pl.ds(tb, TPW)], idx1_v, s0)
    ci2 = pltpu.async_copy(dest_hbm.at[pl.ds(S + tb, TPW)], idx2_v, s1)
    ci1.wait()
    ci2.wait()

    gs = []
    bufs = ((g1a_v, idx1_v, 0, y1_hbm, s0), (g2a_v, idx2_v, 0, y2_hbm, s1),
            (g1b_v, idx1_v, 32, y1_hbm, s2), (g2b_v, idx2_v, 32, y2_hbm, s3))
    for buf, idxv, hoff, _, sem in bufs:
        gs.append(pltpu.async_copy(ffn_hbm.at[idxv.at[pl.ds(hoff, 32)]],
                                   buf, sem))
    ws = []
    for i, (buf, _, hoff, yhbm, sem) in enumerate(bufs):
        gs[i].wait()
        ws.append(pltpu.async_copy(buf, yhbm.at[pl.ds(tb + hoff, 32)], sem))
    for w in ws:
        w.wait()


@functools.cache
def _get_sc_combine():
    return pl.kernel(
        _sc_combine_body,
        out_type=[
            jax.ShapeDtypeStruct((S, D), jnp.float32),
            jax.ShapeDtypeStruct((S, D), jnp.float32),
        ],
        mesh=plsc.VectorSubcoreMesh(core_axis_name="c",
                                    subcore_axis_name="s"),
        compiler_params=pltpu.CompilerParams(needs_layout_passes=False),
        scratch_types=[
            pltpu.VMEM((TPW,), jnp.int32),
            pltpu.VMEM((TPW,), jnp.int32),
            pltpu.VMEM((32, D), jnp.float32),
            pltpu.VMEM((32, D), jnp.float32),
            pltpu.VMEM((32, D), jnp.float32),
            pltpu.VMEM((32, D), jnp.float32),
            pltpu.SemaphoreType.DMA,
            pltpu.SemaphoreType.DMA,
            pltpu.SemaphoreType.DMA,
            pltpu.SemaphoreType.DMA,
        ],
    )


LBLK = 512


def _ln_kernel(y1_ref, y2_ref, v1_ref, v2_ref, x_ref, gamma_ref, beta_ref,
               out_ref):
    y = v1_ref[...] * y1_ref[...] + v2_ref[...] * y2_ref[...] + x_ref[...]
    mean = jnp.mean(y, axis=-1, keepdims=True)
    c = y - mean
    var = jnp.mean(c * c, axis=-1, keepdims=True)
    out_ref[...] = (c * lax.rsqrt(var + 1e-5) * gamma_ref[...]
                    + beta_ref[...])


@jax.jit
def _ln(y1, y2, v1, v2, x2, gamma, beta):
    return pl.pallas_call(
        _ln_kernel,
        grid=(S // LBLK,),
        in_specs=[
            pl.BlockSpec((LBLK, D), lambda i: (i, 0)),
            pl.BlockSpec((LBLK, D), lambda i: (i, 0)),
            pl.BlockSpec((LBLK, 1), lambda i: (i, 0)),
            pl.BlockSpec((LBLK, 1), lambda i: (i, 0)),
            pl.BlockSpec((LBLK, D), lambda i: (i, 0)),
            pl.BlockSpec((1, D), lambda i: (0, 0)),
            pl.BlockSpec((1, D), lambda i: (0, 0)),
        ],
        out_specs=pl.BlockSpec((LBLK, D), lambda i: (i, 0)),
        out_shape=jax.ShapeDtypeStruct((S, D), jnp.float32),
    )(y1, y2, v1, v2, x2, gamma, beta)


def kernel(x, Wg, bg, W1, b1, W2, b2, gamma, beta):
    x2 = x.reshape(S, D)
    dest2, va2, bexp2 = _route(x2, Wg, bg.reshape(1, E))
    dest = dest2.reshape(A)
    bexp = bexp2.reshape(NBLK)
    sx = _get_sc_dispatch()(dest, x2)
    ffn = _ffn(bexp, sx, W1, b1.reshape(E, 1, H),
               W2, b2.reshape(E, 1, D))
    y1, y2 = _get_sc_combine()(dest, ffn)
    out = _ln(y1, y2, va2[:S], va2[S:], x2,
              gamma.reshape(1, D), beta.reshape(1, D))
    return out.reshape(x.shape)


# speedup vs baseline: 1.5506x; 1.4358x over previous
import functools

import jax
import jax.numpy as jnp
from jax import lax
from jax.experimental import pallas as pl
from jax.experimental.pallas import tpu as pltpu
from jax.experimental.pallas import tpu_sc as plsc

D = 768
H = 2048
E = 8
S = 2048
K = 2
A = K * S
BLK = 128
CAP = A + E * BLK
NBLK = CAP // BLK
NC = 2
NS = 16
NW = NC * NS
RPW = CAP // NW
TPW = S // NW
GCH = 32


def _route_kernel(x_ref, wg_ref, bg_ref, dest_ref, va_ref, bexp_ref):
    xb = x_ref[...]
    logits = jnp.dot(xb, wg_ref[...], preferred_element_type=jnp.float32)
    logits = logits + bg_ref[...]
    m = jnp.max(logits, axis=-1, keepdims=True)
    ex = jnp.exp(logits - m)
    rw = ex / jnp.sum(ex, axis=-1, keepdims=True)
    ii = lax.broadcasted_iota(jnp.int32, (S, E), 1)
    m1 = jnp.max(rw, axis=-1, keepdims=True)
    e1 = jnp.min(jnp.where(rw == m1, ii, E), axis=-1, keepdims=True)
    rw2 = jnp.where(ii == e1, -1.0, rw)
    m2 = jnp.max(rw2, axis=-1, keepdims=True)
    e2 = jnp.min(jnp.where(rw2 == m2, ii, E), axis=-1, keepdims=True)

    ea = jnp.concatenate([e1, e2], axis=0)
    va = jnp.concatenate([m1, m2], axis=0)
    va_ref[...] = va

    iiE = lax.broadcasted_iota(jnp.int32, (A, E), 1)
    oh = (ea == iiE).astype(jnp.float32)
    cum = oh
    s = 1
    while s < A:
        cum = cum + jnp.concatenate(
            [jnp.zeros((s, E), jnp.float32), cum[:-s, :]], axis=0)
        s *= 2
    rank = jnp.sum(oh * cum, axis=-1, keepdims=True) - 1.0
    count = cum[A - 1:A, :]
    co = jnp.floor((count + (BLK - 1)) * (1.0 / BLK)) * BLK
    off = jnp.concatenate([jnp.zeros((1, 1), jnp.float32), co[:, :-1]],
                          axis=1)
    s = 1
    while s < E:
        off = off + jnp.concatenate(
            [jnp.zeros((1, s), jnp.float32), off[:, :-s]], axis=1)
        s *= 2
    offsel = jnp.sum(oh * off, axis=-1, keepdims=True)
    dest_ref[...] = (offsel + rank).astype(jnp.int32)

    bb = lax.broadcasted_iota(jnp.int32, (1, NBLK), 1).astype(jnp.float32)
    bb = bb * BLK
    bexpf = jnp.zeros((1, NBLK), jnp.float32)
    for e in range(E):
        bexpf = bexpf + (bb >= off[:, e:e + 1]).astype(jnp.float32)
    bexp_ref[...] = (bexpf - 1.0).astype(jnp.int32)


@jax.jit
def _route(x2, Wg, bg):
    return pl.pallas_call(
        _route_kernel,
        in_specs=[
            pl.BlockSpec((S, D), lambda: (0, 0)),
            pl.BlockSpec((D, E), lambda: (0, 0)),
            pl.BlockSpec((1, E), lambda: (0, 0)),
        ],
        out_specs=[
            pl.BlockSpec((A, 1), lambda: (0, 0)),
            pl.BlockSpec((A, 1), lambda: (0, 0)),
            pl.BlockSpec((1, NBLK), lambda: (0, 0)),
        ],
        out_shape=[
            jax.ShapeDtypeStruct((A, 1), jnp.int32),
            jax.ShapeDtypeStruct((A, 1), jnp.float32),
            jax.ShapeDtypeStruct((1, NBLK), jnp.int32),
        ],
    )(x2, Wg, bg)


def _sc_dispatch_body(dest_hbm, x_hbm, sx_hbm, dest_v, stok_v, rows_v,
                      isem, *gsems):
    cid = lax.axis_index("c")
    sid = lax.axis_index("s")
    wid = sid * NC + cid
    rb = wid * RPW

    with jax.named_scope("disp_copyin"):
        cin0 = pltpu.async_copy(dest_hbm, dest_v, isem)
        lane0 = lax.broadcasted_iota(jnp.int32, (16,), 0)
        for i in range(RPW // 16):
            stok_v[pl.ds(i * 16, 16)] = jnp.bitwise_and(
                rb + i * 16 + lane0, S - 1)
        cin0.wait()

    with jax.named_scope("disp_scatter"):
        lane = lax.broadcasted_iota(jnp.int32, (16,), 0)

        @plsc.parallel_loop(0, A // 16, unroll=8)
        def _(c):
            base = c * 16
            avl = dest_v[pl.ds(base, 16)] - rb
            msk = jnp.logical_and(avl >= 0, avl < RPW)
            tok = jnp.bitwise_and(base + lane, S - 1)
            plsc.store_scatter(stok_v, [avl], tok, mask=msk)

    with jax.named_scope("disp_gather"):
        nst = RPW // 16
        gh = []
        for ch in range(nst):
            gh.append(pltpu.async_copy(
                x_hbm.at[stok_v.at[pl.ds(ch * 16, 16)]],
                rows_v.at[pl.ds(ch * 16, 16)], gsems[ch]))
        wh = []
        for ch in range(nst):
            gh[ch].wait()
            wh.append(pltpu.async_copy(
                rows_v.at[pl.ds(ch * 16, 16)],
                sx_hbm.at[pl.ds(rb + ch * 16, 16)], gsems[ch]))
        for w in wh:
            w.wait()


@functools.cache
def _get_sc_dispatch():
    return pl.kernel(
        _sc_dispatch_body,
        out_type=jax.ShapeDtypeStruct((CAP, D), jnp.float32),
        mesh=plsc.VectorSubcoreMesh(core_axis_name="c",
                                    subcore_axis_name="s"),
        compiler_params=pltpu.CompilerParams(needs_layout_passes=False),
        scratch_types=[
            pltpu.VMEM((A,), jnp.int32),
            pltpu.VMEM((RPW,), jnp.int32),
            pltpu.VMEM((RPW, D), jnp.float32),
        ] + [pltpu.SemaphoreType.DMA] * (1 + RPW // 16),
    )


def _ffn_kernel(meta_ref, sx_ref, b1_ref, b2_ref, w1_hbm, w2_hbm,
                out_ref, w1s_ref, w2s_ref, sem1, sem2):
    i = pl.program_id(0)
    sl = meta_ref[2, i]

    @pl.when(meta_ref[1, i] == 1)
    def _():
        @pl.when(i == 0)
        def _():
            e0 = meta_ref[0, 0]
            pltpu.make_async_copy(w1_hbm.at[e0], w1s_ref.at[0],
                                  sem1.at[0]).start()
            pltpu.make_async_copy(w2_hbm.at[e0], w2s_ref.at[0],
                                  sem2.at[0]).start()

        pltpu.make_async_copy(w1_hbm.at[0], w1s_ref.at[sl],
                              sem1.at[sl]).wait()
        pltpu.make_async_copy(w2_hbm.at[0], w2s_ref.at[sl],
                              sem2.at[sl]).wait()

        @pl.when(meta_ref[3, i] == 1)
        def _():
            ne = meta_ref[4, i]
            pltpu.make_async_copy(w1_hbm.at[ne], w1s_ref.at[1 - sl],
                                  sem1.at[1 - sl]).start()
            pltpu.make_async_copy(w2_hbm.at[ne], w2s_ref.at[1 - sl],
                                  sem2.at[1 - sl]).start()

    xb = sx_ref[...]
    h = jnp.dot(xb, w1s_ref[sl], preferred_element_type=jnp.float32)
    h = jnp.maximum(h + b1_ref[0], 0.0)
    o = jnp.dot(h, w2s_ref[sl], preferred_element_type=jnp.float32)
    out_ref[...] = o + b2_ref[0]


@jax.jit
def _ffn(meta, sx, W1, b1, W2, b2):
    grid_spec = pltpu.PrefetchScalarGridSpec(
        num_scalar_prefetch=1,
        grid=(NBLK,),
        in_specs=[
            pl.BlockSpec((BLK, D), lambda i, m: (i, 0)),
            pl.BlockSpec((1, 1, H), lambda i, m: (m[0, i], 0, 0)),
            pl.BlockSpec((1, 1, D), lambda i, m: (m[0, i], 0, 0)),
            pl.BlockSpec(memory_space=pl.ANY),
            pl.BlockSpec(memory_space=pl.ANY),
        ],
        out_specs=pl.BlockSpec((BLK, D), lambda i, m: (i, 0)),
        scratch_shapes=[
            pltpu.VMEM((2, D, H), jnp.float32),
            pltpu.VMEM((2, H, D), jnp.float32),
            pltpu.SemaphoreType.DMA((2,)),
            pltpu.SemaphoreType.DMA((2,)),
        ],
    )
    return pl.pallas_call(
        _ffn_kernel,
        grid_spec=grid_spec,
        out_shape=jax.ShapeDtypeStruct((CAP, D), jnp.float32),
        compiler_params=pltpu.CompilerParams(
            dimension_semantics=("arbitrary",)),
    )(meta, sx, b1, b2, W1, W2)


def _ffn_meta(bexp):
    pos = jnp.arange(NBLK, dtype=jnp.int32)
    fetch = jnp.concatenate(
        [jnp.ones((1,), jnp.int32),
         (bexp[1:] != bexp[:-1]).astype(jnp.int32)])
    runid = jnp.cumsum(fetch) - 1
    slot = (runid % 2).astype(jnp.int32)
    start_idx = jnp.where(fetch == 1, pos, 2 * NBLK)
    rev = lax.cummin(start_idx[::-1])[::-1]
    nxt_start = jnp.concatenate(
        [rev[1:], jnp.full((1,), 2 * NBLK, jnp.int32)])
    has_next = (nxt_start < NBLK).astype(jnp.int32)
    nre = bexp[jnp.clip(nxt_start, 0, NBLK - 1)]
    return jnp.stack([bexp, fetch, slot, has_next, nre])


def _sc_combine_body(dest_hbm, ffn_hbm, y1_hbm, y2_hbm,
                     idx_v, g_v, isem, *gsems):
    cid = lax.axis_index("c")
    sid = lax.axis_index("s")
    wid = sid * NC + cid
    tb = wid * TPW

    ci1 = pltpu.async_copy(dest_hbm.at[pl.ds(tb, TPW)],
                           idx_v.at[pl.ds(0, TPW)], isem)
    ci2 = pltpu.async_copy(dest_hbm.at[pl.ds(S + tb, TPW)],
                           idx_v.at[pl.ds(TPW, TPW)], isem)
    ci1.wait()
    ci2.wait()

    nst = 2 * TPW // 16
    gh = []
    for ch in range(nst):
        gh.append(pltpu.async_copy(
            ffn_hbm.at[idx_v.at[pl.ds(ch * 16, 16)]],
            g_v.at[pl.ds(ch * 16, 16)], gsems[ch]))
    wh = []
    for ch in range(nst):
        gh[ch].wait()
        yhbm = y1_hbm if ch < TPW // 16 else y2_hbm
        toff = tb + (ch * 16) % TPW
        wh.append(pltpu.async_copy(
            g_v.at[pl.ds(ch * 16, 16)], yhbm.at[pl.ds(toff, 16)],
            gsems[ch]))
    for w in wh:
        w.wait()


@functools.cache
def _get_sc_combine():
    return pl.kernel(
        _sc_combine_body,
        out_type=[
            jax.ShapeDtypeStruct((S, D), jnp.float32),
            jax.ShapeDtypeStruct((S, D), jnp.float32),
        ],
        mesh=plsc.VectorSubcoreMesh(core_axis_name="c",
                                    subcore_axis_name="s"),
        compiler_params=pltpu.CompilerParams(needs_layout_passes=False),
        scratch_types=[
            pltpu.VMEM((2 * TPW,), jnp.int32),
            pltpu.VMEM((2 * TPW, D), jnp.float32),
        ] + [pltpu.SemaphoreType.DMA] * (1 + 2 * TPW // 16),
    )


LBLK = 512


def _ln_kernel(y1_ref, y2_ref, v1_ref, v2_ref, x_ref, gamma_ref, beta_ref,
               out_ref):
    y = v1_ref[...] * y1_ref[...] + v2_ref[...] * y2_ref[...] + x_ref[...]
    mean = jnp.mean(y, axis=-1, keepdims=True)
    c = y - mean
    var = jnp.mean(c * c, axis=-1, keepdims=True)
    out_ref[...] = (c * lax.rsqrt(var + 1e-5) * gamma_ref[...]
                    + beta_ref[...])


@jax.jit
def _ln(y1, y2, v1, v2, x2, gamma, beta):
    return pl.pallas_call(
        _ln_kernel,
        grid=(S // LBLK,),
        in_specs=[
            pl.BlockSpec((LBLK, D), lambda i: (i, 0)),
            pl.BlockSpec((LBLK, D), lambda i: (i, 0)),
            pl.BlockSpec((LBLK, 1), lambda i: (i, 0)),
            pl.BlockSpec((LBLK, 1), lambda i: (i, 0)),
            pl.BlockSpec((LBLK, D), lambda i: (i, 0)),
            pl.BlockSpec((1, D), lambda i: (0, 0)),
            pl.BlockSpec((1, D), lambda i: (0, 0)),
        ],
        out_specs=pl.BlockSpec((LBLK, D), lambda i: (i, 0)),
        out_shape=jax.ShapeDtypeStruct((S, D), jnp.float32),
    )(y1, y2, v1, v2, x2, gamma, beta)


def kernel(x, Wg, bg, W1, b1, W2, b2, gamma, beta):
    x2 = x.reshape(S, D)
    dest2, va2, bexp2 = _route(x2, Wg, bg.reshape(1, E))
    dest = dest2.reshape(A)
    bexp = bexp2.reshape(NBLK)
    sx = _get_sc_dispatch()(dest, x2)
    ffn = _ffn(_ffn_meta(bexp), sx, W1, b1.reshape(E, 1, H),
               W2, b2.reshape(E, 1, D))
    y1, y2 = _get_sc_combine()(dest, ffn)
    out = _ln(y1, y2, va2[:S], va2[S:], x2,
              gamma.reshape(1, D), beta.reshape(1, D))
    return out.reshape(x.shape)
